# baseline (speedup 1.0000x reference)
import functools

import jax
import jax.numpy as jnp
from jax import lax
from jax.experimental import pallas as pl
from jax.experimental.pallas import tpu as pltpu
from jax.experimental.pallas import tpu_sc as plsc

B = 16384
D_ACCT = 64
D_TIME = 16
D_OUT = 96
L = 16

NC = 2
NS = 16
NW = NC * NS
BPW = B // NW

_mesh = plsc.VectorSubcoreMesh(core_axis_name="c", subcore_axis_name="s")


@functools.partial(
    pl.kernel,
    mesh=_mesh,
    out_type=jax.ShapeDtypeStruct((B * D_OUT,), jnp.float32),
    scratch_types=[
        pltpu.VMEM((BPW,), jnp.int32),
        pltpu.VMEM((24 * D_TIME,), jnp.float32),
        pltpu.VMEM((7 * D_TIME,), jnp.float32),
        pltpu.VMEM((BPW * D_ACCT,), jnp.float32),
        pltpu.VMEM((BPW * D_OUT,), jnp.float32),
        pltpu.SemaphoreType.DMA,
    ],
    compiler_params=pltpu.CompilerParams(needs_layout_passes=False),
)
def _sc_embed(packed_hbm, hour_tab_hbm, wday_tab_hbm, acct_tab_hbm, out_hbm,
              packed_v, hour_v, wday_v, acct_v, out_v, sem):
    wid = lax.axis_index("s") * NC + lax.axis_index("c")
    base = wid * BPW

    pltpu.sync_copy(packed_hbm.at[wid], packed_v)
    pltpu.sync_copy(hour_tab_hbm, hour_v)
    pltpu.sync_copy(wday_tab_hbm, wday_v)

    lane = lax.broadcasted_iota(jnp.int32, (L,), 0)

    def fetch(r, _):
        vec = packed_v[pl.ds((r >> 4) << 4, L)]
        p = jnp.sum(jnp.where(lane == (r & (L - 1)), vec, 0))
        a = pl.multiple_of((p & 0xFFFFF) * D_ACCT, 8)
        pltpu.async_copy(acct_tab_hbm.at[pl.ds(a, D_ACCT)],
                         acct_v.at[pl.ds(r * D_ACCT, D_ACCT)], sem)
        return 0

    lax.fori_loop(0, BPW, fetch, 0, unroll=8)
    pltpu.make_async_copy(acct_tab_hbm.at[pl.ds(0, BPW * D_ACCT)],
                          acct_v, sem).wait()

    def assemble(r, _):
        for k in range(D_ACCT // L):
            out_v[pl.ds(r * D_OUT + k * L, L)] = acct_v[pl.ds(r * D_ACCT + k * L, L)]
        return 0

    lax.fori_loop(0, BPW, assemble, 0, unroll=4)

    def towers(g, _):
        i0 = g * L
        vec = packed_v[pl.ds(i0, L)]
        hrow = ((vec >> 20) & 31) * D_TIME
        wrow = ((vec >> 25) & 7) * D_TIME
        opos = (i0 + lane) * D_OUT
        for j in range(D_TIME):
            hv = plsc.load_gather(hour_v, [hrow + j])
            plsc.store_scatter(out_v, [opos + (D_ACCT + j)], hv)
            wv = plsc.load_gather(wday_v, [wrow + j])
            plsc.store_scatter(out_v, [opos + (D_ACCT + D_TIME + j)], wv)
        return 0

    lax.fori_loop(0, BPW // L, towers, 0)

    pltpu.sync_copy(out_v, out_hbm.at[pl.ds(base * D_OUT, BPW * D_OUT)])


def kernel(account_id, order_hour, order_weekday, account_table, hour_table, weekday_table):
    aid = account_id.astype(jnp.int32)
    packed = (aid | (order_hour.astype(jnp.int32) << 20)
              | (order_weekday.astype(jnp.int32) << 25)).reshape(NW, BPW)
    out = _sc_embed(packed, hour_table.reshape(-1), weekday_table.reshape(-1),
                    account_table.reshape(-1))
    return out.reshape(B, D_OUT)


# speedup vs baseline: 1.6149x; 1.6149x over previous
import functools

import jax
import jax.numpy as jnp
from jax import lax
from jax.experimental import pallas as pl
from jax.experimental.pallas import tpu as pltpu
from jax.experimental.pallas import tpu_sc as plsc

B = 16384
D_ACCT = 64
D_TIME = 16
D_OUT = 96
L = 16

NC = 2
NS = 16
NW = NC * NS
BPW = B // NW

_mesh = plsc.VectorSubcoreMesh(core_axis_name="c", subcore_axis_name="s")


@functools.partial(
    pl.kernel,
    mesh=_mesh,
    out_type=jax.ShapeDtypeStruct((B * D_OUT,), jnp.float32),
    scratch_types=[
        pltpu.VMEM((BPW,), jnp.int32),
        pltpu.VMEM((24 * D_TIME,), jnp.float32),
        pltpu.VMEM((7 * D_TIME,), jnp.float32),
        pltpu.VMEM((BPW, D_ACCT), jnp.float32),
        pltpu.VMEM((BPW * D_OUT,), jnp.float32),
        pltpu.SemaphoreType.DMA,
    ],
    compiler_params=pltpu.CompilerParams(needs_layout_passes=False),
)
def _sc_embed(packed_hbm, hour_tab_hbm, wday_tab_hbm, acct_tab_hbm, out_hbm,
              packed_v, hour_v, wday_v, acct_v, out_v, sem):
    wid = lax.axis_index("s") * NC + lax.axis_index("c")
    base = wid * BPW

    pltpu.sync_copy(packed_hbm.at[wid], packed_v)
    pltpu.sync_copy(hour_tab_hbm, hour_v)
    pltpu.sync_copy(wday_tab_hbm, wday_v)

    lane = lax.broadcasted_iota(jnp.int32, (L,), 0)

    def fetch(r, _):
        vec = packed_v[pl.ds((r >> 4) << 4, L)]
        p = jnp.sum(jnp.where(lane == (r & (L - 1)), vec, 0))
        a = p & 0xFFFFF
        pltpu.async_copy(acct_tab_hbm.at[a], acct_v.at[r], sem)
        return 0

    lax.fori_loop(0, BPW, fetch, 0, unroll=4)
    pltpu.make_async_copy(acct_tab_hbm.at[pl.ds(0, BPW)], acct_v, sem).wait()

    def assemble(r, _):
        for k in range(D_ACCT // L):
            out_v[pl.ds(r * D_OUT + k * L, L)] = acct_v[r, pl.ds(k * L, L)]
        return 0

    lax.fori_loop(0, BPW, assemble, 0, unroll=4)

    def towers(g, _):
        i0 = g * L
        vec = packed_v[pl.ds(i0, L)]
        hrow = ((vec >> 20) & 31) * D_TIME
        wrow = ((vec >> 25) & 7) * D_TIME
        opos = (i0 + lane) * D_OUT
        for j in range(D_TIME):
            hv = plsc.load_gather(hour_v, [hrow + j])
            plsc.store_scatter(out_v, [opos + (D_ACCT + j)], hv)
            wv = plsc.load_gather(wday_v, [wrow + j])
            plsc.store_scatter(out_v, [opos + (D_ACCT + D_TIME + j)], wv)
        return 0

    lax.fori_loop(0, BPW // L, towers, 0)

    pltpu.sync_copy(out_v, out_hbm.at[pl.ds(base * D_OUT, BPW * D_OUT)])


def kernel(account_id, order_hour, order_weekday, account_table, hour_table, weekday_table):
    aid = account_id.astype(jnp.int32)
    packed = (aid | (order_hour.astype(jnp.int32) << 20)
              | (order_weekday.astype(jnp.int32) << 25)).reshape(NW, BPW)
    out = _sc_embed(packed, hour_table.reshape(-1), weekday_table.reshape(-1),
                    account_table)
    return out.reshape(B, D_OUT)
